# baseline (speedup 1.0000x reference)
import functools

import numpy as np
import jax
import jax.numpy as jnp
from jax import lax
from jax.experimental import pallas as pl
from jax.experimental.pallas import tpu as pltpu
from jax.experimental.pallas import tpu_sc as plsc

_N_ROIS = 20000
_N_GT = 64
_N = _N_ROIS + _N_GT
_NSUB = 16
_RBLK = 512
_NPAD = 40 * _RBLK
_CHUNK2 = _NPAD // _NSUB
_SBUF = 2048
_MARGIN = 80
_SLEN = 1 << 18

_STREAM_NP = np.random.RandomState(111).randint(
    0, 2**32, size=_SLEN, dtype=np.uint32).view(np.int32)


def _iou_tc_body(rois_ref, bbox_ref, max_ref, packed_ref):
    i = pl.program_id(0)
    r = rois_ref[...]
    rx1 = r[:, 0:1]
    ry1 = r[:, 1:2]
    rx2 = r[:, 2:3]
    ry2 = r[:, 3:4]
    gx1 = bbox_ref[0:1, :]
    gy1 = bbox_ref[1:2, :]
    gx2 = bbox_ref[2:3, :]
    gy2 = bbox_ref[3:4, :]
    ltx = jnp.maximum(rx1, gx1)
    lty = jnp.maximum(ry1, gy1)
    rbx = jnp.minimum(rx2, gx2)
    rby = jnp.minimum(ry2, gy2)
    wx = jnp.maximum(rbx - ltx, 0.0)
    wy = jnp.maximum(rby - lty, 0.0)
    inter = wx * wy
    area_r = (rx2 - rx1) * (ry2 - ry1)
    area_g = (gx2 - gx1) * (gy2 - gy1)
    iou = inter / (area_r + area_g - inter)
    m = jnp.max(iou, axis=1, keepdims=True)
    ids = lax.broadcasted_iota(jnp.int32, (_RBLK, _N_GT), 1)
    am = jnp.min(jnp.where(iou == m, ids, _N_GT), axis=1, keepdims=True)
    row = i * _RBLK + lax.broadcasted_iota(jnp.int32, (_RBLK, 1), 0)
    valid = row < _N
    max_ref[...] = jnp.where(valid, m, -1.0)
    packed_ref[:, 0:4] = r
    packed_ref[:, 4:5] = jnp.where(valid, am, 0).astype(jnp.float32)
    packed_ref[:, 5:16] = jnp.zeros((_RBLK, 11), jnp.float32)


_iou_call = pl.pallas_call(
    _iou_tc_body,
    grid=(_NPAD // _RBLK,),
    in_specs=[
        pl.BlockSpec((_RBLK, 4), lambda i: (i, 0)),
        pl.BlockSpec((4, _N_GT), lambda i: (0, 0)),
    ],
    out_specs=[
        pl.BlockSpec((_RBLK, 1), lambda i: (i, 0)),
        pl.BlockSpec((_RBLK, 16), lambda i: (i, 0)),
    ],
    out_shape=[
        jax.ShapeDtypeStruct((_NPAD, 1), jnp.float32),
        jax.ShapeDtypeStruct((_NPAD, 16), jnp.float32),
    ],
)


def _splat(x):
    return jnp.full((16,), x, jnp.int32)


def _sread1(ref, i):
    return plsc.load_gather(ref, [_splat(0) + i])[0]


def _ln(x):
    bits = plsc.bitcast(x, jnp.int32)
    e = (bits >> 23) - 127
    m = plsc.bitcast((bits & 0x007FFFFF) | 0x3F800000, jnp.float32)
    big = m > 1.4142135623730951
    m = jnp.where(big, m * 0.5, m)
    e = jnp.where(big, e + 1, e)
    t = (m - 1.0) / (m + 1.0)
    t2 = t * t
    s = t * (2.0 + t2 * (2.0 / 3.0 + t2 * (2.0 / 5.0
                                           + t2 * (2.0 / 7.0
                                                   + t2 * (2.0 / 9.0)))))
    return e.astype(jnp.float32) * 0.6931471805599453 + s


_mesh = plsc.VectorSubcoreMesh(core_axis_name="c", subcore_axis_name="s")


@functools.partial(
    pl.kernel,
    out_type=[
        jax.ShapeDtypeStruct((128,), jnp.int32),
        jax.ShapeDtypeStruct((128, 4), jnp.float32),
        jax.ShapeDtypeStruct((128, 4), jnp.float32),
    ],
    mesh=_mesh,
    compiler_params=pltpu.CompilerParams(
        needs_layout_passes=False, use_tc_tiling_on_sc=False),
    scratch_types=[
        pltpu.VMEM((_CHUNK2,), jnp.float32),
        pltpu.VMEM((_CHUNK2,), jnp.int32),
        pltpu.VMEM((_CHUNK2,), jnp.int32),
        pltpu.VMEM((16,), jnp.int32),
        pltpu.VMEM_SHARED((_NSUB, _CHUNK2), jnp.int32),
        pltpu.VMEM_SHARED((_NSUB, _CHUNK2), jnp.int32),
        pltpu.VMEM_SHARED((_NSUB, 16), jnp.int32),
        pltpu.VMEM((_NSUB, 16), jnp.int32),
        pltpu.VMEM((_NSUB, _CHUNK2), jnp.int32),
        pltpu.VMEM((_NSUB, _CHUNK2), jnp.int32),
        pltpu.VMEM((_SBUF,), jnp.int32),
        pltpu.VMEM((_NPAD,), jnp.int32),
        pltpu.VMEM((_NSUB,), jnp.int32),
        pltpu.VMEM((_NSUB,), jnp.int32),
        pltpu.VMEM((_NSUB,), jnp.int32),
        pltpu.VMEM((_NSUB,), jnp.int32),
        pltpu.VMEM((128,), jnp.int32),
        pltpu.VMEM((128,), jnp.int32),
        pltpu.VMEM((_N_GT, 4), jnp.float32),
        pltpu.VMEM((_N_GT,), jnp.int32),
        pltpu.VMEM((128, 16), jnp.float32),
        pltpu.VMEM((128, 4), jnp.float32),
        pltpu.VMEM((128, 4), jnp.float32),
        pltpu.SemaphoreType.DMA,
    ],
)
def _fused_call(iou_hbm, stream_hbm, packed_hbm, bbox_hbm, label_hbm,
                cls_hbm, reg_hbm, srois_hbm,
                iou_v, posloc_v, negloc_v, cnt16_v, pos_sh, neg_sh, cnt_sh,
                cnt_v, posl_v, negl_v, sbuf_v, perm_v, ipos_v,
                epos_v, ineg_v, eneg_v, keep_v, cls_v, bbox_v, label_v,
                roisg_v, reg_v, srois_v, sem):
    core = lax.axis_index("c")
    sub = lax.axis_index("s")

    @pl.when(core == 0)
    def _():
        base = sub * _CHUNK2
        pltpu.sync_copy(iou_hbm.at[pl.ds(base, _CHUNK2)], iou_v)
        iota = lax.iota(jnp.int32, 16)
        pos_cnt = jnp.int32(0)
        neg_cnt = jnp.int32(0)
        for t in range(_CHUNK2 // 16):
            v = iou_v[pl.ds(t * 16, 16)]
            gidx = base + t * 16 + iota
            pm = v >= 0.5
            nm = jnp.logical_and(v < 0.5, v >= 0.0)
            ppos = pos_cnt + plsc.cumsum(pm.astype(jnp.int32)) - 1
            plsc.store_scatter(posloc_v, [ppos], gidx, mask=pm)
            pos_cnt = pos_cnt + jnp.sum(pm.astype(jnp.int32))
            npos = neg_cnt + plsc.cumsum(nm.astype(jnp.int32)) - 1
            plsc.store_scatter(negloc_v, [npos], gidx, mask=nm)
            neg_cnt = neg_cnt + jnp.sum(nm.astype(jnp.int32))
        cnt16_v[...] = jnp.where(iota == 0, pos_cnt,
                                 jnp.where(iota == 1, neg_cnt, 0))
        pltpu.sync_copy(posloc_v, pos_sh.at[sub])
        pltpu.sync_copy(negloc_v, neg_sh.at[sub])
        pltpu.sync_copy(cnt16_v, cnt_sh.at[sub])

    plsc.subcore_barrier()

    @pl.when(jnp.logical_and(core == 0, sub == 0))
    def _():
        pltpu.sync_copy(cnt_sh, cnt_v)
        pltpu.sync_copy(pos_sh, posl_v)
        pltpu.sync_copy(neg_sh, negl_v)
        pltpu.sync_copy(bbox_hbm, bbox_v)
        pltpu.sync_copy(label_hbm, label_v)
        pltpu.sync_copy(stream_hbm.at[pl.ds(0, _SBUF)], sbuf_v)

        iota = lax.iota(jnp.int32, 16)

        def prefixes(col, i_ref, e_ref):
            c0 = plsc.load_gather(cnt_v, [iota, _splat(col)])
            i0 = plsc.cumsum(c0)
            i_ref[...] = i0
            e_ref[...] = i0 - c0
            return i0[15]

        pop_pos = prefixes(0, ipos_v, epos_v)
        pop_neg = prefixes(1, ineg_v, eneg_v)

        n_init = (jnp.maximum(pop_pos, 32) + 15) // 16

        def init_body(t, c):
            b = t * 16
            plsc.store_scatter(perm_v, [b + iota], b + iota)
            return c
        lax.fori_loop(0, n_init, init_body, 0)

        def smear(x):
            x = x | (x >> 1)
            x = x | (x >> 2)
            x = x | (x >> 4)
            x = x | (x >> 8)
            x = x | (x >> 16)
            return x

        def ensure(ptr, base):
            def do(_):
                nb = jnp.minimum((ptr // 8) * 8, _SLEN - _SBUF)
                nb = pl.multiple_of(nb, 8)
                pltpu.sync_copy(stream_hbm.at[pl.ds(nb, _SBUF)], sbuf_v)
                return nb
            need = jnp.logical_or(ptr < base, ptr + _MARGIN > base + _SBUF)
            return lax.cond(need, do, lambda _: base, 0)

        def draw(ptr, base, bound_u):
            mask = smear(bound_u)

            def cond(c):
                return c[0] > bound_u

            def body(c):
                _, p = c
                off = jnp.minimum(p - base, _SBUF - 1)
                w = _sread1(sbuf_v, off).astype(jnp.uint32)
                return (w & mask, p + 1)

            v, ptr = lax.while_loop(cond, body, (bound_u + jnp.uint32(1), ptr))
            return v, ptr

        lane01 = iota < 2

        def fy_body(d, carry):
            ptr, base = carry
            i = pop_pos - 1 - d
            base = ensure(ptr, base)
            j_u, ptr = draw(ptr, base, i.astype(jnp.uint32))
            j = j_u.astype(jnp.int32)
            ij = jnp.where(iota == 0, i, j)
            pij = plsc.load_gather(perm_v, [ij])
            pi = pij[0]
            pj = pij[1]
            plsc.store_scatter(perm_v, [ij],
                               jnp.where(iota == 0, pj, pi), mask=lane01)
            return (ptr, base)

        _, base_end = lax.fori_loop(
            0, jnp.maximum(pop_pos - 1, 0), fy_body,
            (jnp.int32(0), jnp.int32(0)))

        def vlookup(list_ref, i_ref, e_ref, r):
            w = jnp.zeros((16,), jnp.int32)
            for step in (8, 4, 2, 1):
                probe = jnp.minimum(w + (step - 1), _NSUB - 1)
                ipv = plsc.load_gather(i_ref, [probe])
                w = jnp.where(ipv <= r, w + step, w)
            w = jnp.minimum(w, _NSUB - 1)
            off = jnp.clip(r - plsc.load_gather(e_ref, [w]), 0, _CHUNK2 - 1)
            return plsc.load_gather(list_ref, [w, off])

        for c in range(2):
            r = perm_v[pl.ds(c * 16, 16)]
            pidx = vlookup(posl_v, ipos_v, epos_v, r)
            nidx = vlookup(negl_v, ineg_v, eneg_v, r - pop_pos)
            keep_v[pl.ds(c * 16, 16)] = jnp.where(r < pop_pos, pidx, nidx)

        bound_n = (pop_neg - 1).astype(jnp.uint32)
        maskb = smear(bound_n)

        def ncond(c):
            return c[0] < 96

        def nbody(c):
            cnt, ptr, base = c
            base = ensure(ptr, base)
            off = jnp.minimum(ptr - base, _SBUF - 16)
            w = plsc.bitcast(plsc.load_gather(sbuf_v, [off + iota]),
                             jnp.uint32)
            wm = w & maskb
            ok = wm <= bound_n
            oki = ok.astype(jnp.int32)
            pos = cnt + plsc.cumsum(oki) - 1
            plsc.store_scatter(
                keep_v, [jnp.clip(32 + pos, 32, 127)],
                plsc.bitcast(wm, jnp.int32),
                mask=jnp.logical_and(ok, pos < 96))
            return (cnt + jnp.sum(oki), ptr + 16, base)

        lax.while_loop(ncond, nbody, (jnp.int32(0), jnp.int32(0), base_end))

        for c in range(2, 8):
            d = keep_v[pl.ds(c * 16, 16)]
            idx = vlookup(negl_v, ineg_v, eneg_v, d)
            keep_v[pl.ds(c * 16, 16)] = jnp.where(
                _splat(pop_neg) == 0, 0, idx)

        pltpu.async_copy(packed_hbm.at[keep_v], roisg_v, sem).wait()
        for c in range(8):
            rows = c * 16 + iota
            px1 = plsc.load_gather(roisg_v, [rows, _splat(0)])
            py1 = plsc.load_gather(roisg_v, [rows, _splat(1)])
            px2 = plsc.load_gather(roisg_v, [rows, _splat(2)])
            py2 = plsc.load_gather(roisg_v, [rows, _splat(3)])
            avec = plsc.load_gather(roisg_v, [rows, _splat(4)]).astype(
                jnp.int32)
            lvec = plsc.load_gather(label_v, [avec])
            if c < 2:
                cls_v[pl.ds(c * 16, 16)] = lvec + 1
            else:
                cls_v[pl.ds(c * 16, 16)] = jnp.zeros((16,), jnp.int32)
            gx1 = plsc.load_gather(bbox_v, [avec, _splat(0)])
            gy1 = plsc.load_gather(bbox_v, [avec, _splat(1)])
            gx2 = plsc.load_gather(bbox_v, [avec, _splat(2)])
            gy2 = plsc.load_gather(bbox_v, [avec, _splat(3)])
            pw = px2 - px1
            ph = py2 - py1
            tx = ((gx2 + gx1) / 2.0 - (px2 + px1) / 2.0) / pw
            ty = ((gy2 + gy1) / 2.0 - (py2 + py1) / 2.0) / ph
            tw = _ln((gx2 - gx1) / pw)
            th = _ln((gy2 - gy1) / ph)
            for col, vec in ((0, px1), (1, py1), (2, px2), (3, py2)):
                plsc.store_scatter(srois_v, [rows, _splat(col)], vec)
            for col, vec in ((0, tx), (1, ty), (2, tw), (3, th)):
                plsc.store_scatter(reg_v, [rows, _splat(col)], vec)

        pltpu.sync_copy(cls_v, cls_hbm)
        pltpu.sync_copy(reg_v, reg_hbm)
        pltpu.sync_copy(srois_v, srois_hbm)


def kernel(bbox, label, rois):
    bbox0 = bbox[0]
    label0 = label[0]
    rois_cat = jnp.concatenate([rois, bbox0], axis=0)
    rois_pad = jnp.pad(rois_cat, ((0, _NPAD - _N), (0, 0)))
    iou_max2, packed = _iou_call(rois_pad, bbox0.T)
    iou_max = iou_max2.reshape(_NPAD)
    stream = jnp.asarray(_STREAM_NP)
    cls, reg, sample_rois = _fused_call(
        iou_max, stream, packed, bbox0, label0)
    return (cls, reg, sample_rois)


# speedup vs baseline: 1.1252x; 1.1252x over previous
import functools

import numpy as np
import jax
import jax.numpy as jnp
from jax import lax
from jax.experimental import pallas as pl
from jax.experimental.pallas import tpu as pltpu
from jax.experimental.pallas import tpu_sc as plsc

_N_ROIS = 20000
_N_GT = 64
_N = _N_ROIS + _N_GT
_NSUB = 16
_RBLK = 512
_NPAD = 40 * _RBLK
_CHUNK2 = _NPAD // _NSUB
_SBUF = 2048
_MARGIN = 80
_SLEN = 1 << 18

_STREAM_NP = np.random.RandomState(111).randint(
    0, 2**32, size=_SLEN, dtype=np.uint32).view(np.int32)


def _iou_tc_body(rois_ref, bbox_ref, max_ref, amax_ref):
    i = pl.program_id(0)
    rx1 = rois_ref[0:1, :]
    ry1 = rois_ref[1:2, :]
    rx2 = rois_ref[2:3, :]
    ry2 = rois_ref[3:4, :]
    gx1 = bbox_ref[:, 0:1]
    gy1 = bbox_ref[:, 1:2]
    gx2 = bbox_ref[:, 2:3]
    gy2 = bbox_ref[:, 3:4]
    ltx = jnp.maximum(rx1, gx1)
    lty = jnp.maximum(ry1, gy1)
    rbx = jnp.minimum(rx2, gx2)
    rby = jnp.minimum(ry2, gy2)
    wx = jnp.maximum(rbx - ltx, 0.0)
    wy = jnp.maximum(rby - lty, 0.0)
    inter = wx * wy
    area_r = (rx2 - rx1) * (ry2 - ry1)
    area_g = (gx2 - gx1) * (gy2 - gy1)
    iou = inter / (area_r + area_g - inter)
    m = jnp.max(iou, axis=0, keepdims=True)
    ids = lax.broadcasted_iota(jnp.int32, (_N_GT, _RBLK), 0)
    am = jnp.min(jnp.where(iou == m, ids, _N_GT), axis=0, keepdims=True)
    col = i * _RBLK + lax.broadcasted_iota(jnp.int32, (1, _RBLK), 1)
    valid = col < _N
    max_ref[...] = jnp.where(valid, m, -1.0)
    amax_ref[...] = jnp.where(valid, am, 0)


_iou_call = pl.pallas_call(
    _iou_tc_body,
    grid=(_NPAD // _RBLK,),
    in_specs=[
        pl.BlockSpec((4, _RBLK), lambda i: (0, i)),
        pl.BlockSpec((_N_GT, 4), lambda i: (0, 0)),
    ],
    out_specs=[
        pl.BlockSpec((1, _RBLK), lambda i: (0, i)),
        pl.BlockSpec((1, _RBLK), lambda i: (0, i)),
    ],
    out_shape=[
        jax.ShapeDtypeStruct((1, _NPAD), jnp.float32),
        jax.ShapeDtypeStruct((1, _NPAD), jnp.int32),
    ],
)


def _splat(x):
    return jnp.full((16,), x, jnp.int32)


def _sread1(ref, i):
    return plsc.load_gather(ref, [_splat(0) + i])[0]


def _ln(x):
    bits = plsc.bitcast(x, jnp.int32)
    e = (bits >> 23) - 127
    m = plsc.bitcast((bits & 0x007FFFFF) | 0x3F800000, jnp.float32)
    big = m > 1.4142135623730951
    m = jnp.where(big, m * 0.5, m)
    e = jnp.where(big, e + 1, e)
    t = (m - 1.0) / (m + 1.0)
    t2 = t * t
    s = t * (2.0 + t2 * (2.0 / 3.0 + t2 * (2.0 / 5.0
                                           + t2 * (2.0 / 7.0
                                                   + t2 * (2.0 / 9.0)))))
    return e.astype(jnp.float32) * 0.6931471805599453 + s


_mesh = plsc.VectorSubcoreMesh(core_axis_name="c", subcore_axis_name="s")


@functools.partial(
    pl.kernel,
    out_type=[
        jax.ShapeDtypeStruct((128,), jnp.int32),
        jax.ShapeDtypeStruct((128, 4), jnp.float32),
        jax.ShapeDtypeStruct((128, 4), jnp.float32),
    ],
    mesh=_mesh,
    compiler_params=pltpu.CompilerParams(
        needs_layout_passes=False, use_tc_tiling_on_sc=False),
    scratch_types=[
        pltpu.VMEM((_CHUNK2,), jnp.float32),
        pltpu.VMEM((_CHUNK2,), jnp.int32),
        pltpu.VMEM((_CHUNK2,), jnp.int32),
        pltpu.VMEM((16,), jnp.int32),
        pltpu.VMEM_SHARED((_NSUB, _CHUNK2), jnp.int32),
        pltpu.VMEM_SHARED((_NSUB, _CHUNK2), jnp.int32),
        pltpu.VMEM_SHARED((_NSUB, 16), jnp.int32),
        pltpu.VMEM((_NSUB, 16), jnp.int32),
        pltpu.VMEM((_NSUB, _CHUNK2), jnp.int32),
        pltpu.VMEM((_NSUB, _CHUNK2), jnp.int32),
        pltpu.VMEM((_NPAD,), jnp.int32),
        pltpu.VMEM((_SBUF,), jnp.int32),
        pltpu.VMEM((_NPAD,), jnp.int32),
        pltpu.VMEM((_NSUB,), jnp.int32),
        pltpu.VMEM((_NSUB,), jnp.int32),
        pltpu.VMEM((_NSUB,), jnp.int32),
        pltpu.VMEM((_NSUB,), jnp.int32),
        pltpu.VMEM((128,), jnp.int32),
        pltpu.VMEM((128,), jnp.int32),
        pltpu.VMEM((_N_GT, 4), jnp.float32),
        pltpu.VMEM((_N_GT,), jnp.int32),
        pltpu.VMEM((128, 16), jnp.float32),
        pltpu.VMEM((128, 4), jnp.float32),
        pltpu.VMEM((128, 4), jnp.float32),
        pltpu.SemaphoreType.DMA,
    ],
)
def _fused_call(iou_hbm, amax_hbm, stream_hbm, rois16_hbm, bbox_hbm,
                label_hbm,
                cls_hbm, reg_hbm, srois_hbm,
                iou_v, posloc_v, negloc_v, cnt16_v, pos_sh, neg_sh, cnt_sh,
                cnt_v, posl_v, negl_v, amax_v, sbuf_v, perm_v, ipos_v,
                epos_v, ineg_v, eneg_v, keep_v, cls_v, bbox_v, label_v,
                roisg_v, reg_v, srois_v, sem):
    core = lax.axis_index("c")
    sub = lax.axis_index("s")

    @pl.when(core == 0)
    def _():
        base = sub * _CHUNK2
        pltpu.sync_copy(iou_hbm.at[pl.ds(base, _CHUNK2)], iou_v)
        iota = lax.iota(jnp.int32, 16)
        pos_cnt = jnp.int32(0)
        neg_cnt = jnp.int32(0)
        for t in range(_CHUNK2 // 16):
            v = iou_v[pl.ds(t * 16, 16)]
            gidx = base + t * 16 + iota
            pm = v >= 0.5
            nm = jnp.logical_and(v < 0.5, v >= 0.0)
            ppos = pos_cnt + plsc.cumsum(pm.astype(jnp.int32)) - 1
            plsc.store_scatter(posloc_v, [ppos], gidx, mask=pm)
            pos_cnt = pos_cnt + jnp.sum(pm.astype(jnp.int32))
            npos = neg_cnt + plsc.cumsum(nm.astype(jnp.int32)) - 1
            plsc.store_scatter(negloc_v, [npos], gidx, mask=nm)
            neg_cnt = neg_cnt + jnp.sum(nm.astype(jnp.int32))
        cnt16_v[...] = jnp.where(iota == 0, pos_cnt,
                                 jnp.where(iota == 1, neg_cnt, 0))
        pltpu.sync_copy(posloc_v, pos_sh.at[sub])
        pltpu.sync_copy(negloc_v, neg_sh.at[sub])
        pltpu.sync_copy(cnt16_v, cnt_sh.at[sub])

    plsc.subcore_barrier()

    @pl.when(jnp.logical_and(core == 0, sub == 0))
    def _():
        pltpu.sync_copy(cnt_sh, cnt_v)
        pltpu.sync_copy(pos_sh, posl_v)
        pltpu.sync_copy(neg_sh, negl_v)
        pltpu.sync_copy(amax_hbm, amax_v)
        pltpu.sync_copy(bbox_hbm, bbox_v)
        pltpu.sync_copy(label_hbm, label_v)
        pltpu.sync_copy(stream_hbm.at[pl.ds(0, _SBUF)], sbuf_v)

        iota = lax.iota(jnp.int32, 16)

        def prefixes(col, i_ref, e_ref):
            c0 = plsc.load_gather(cnt_v, [iota, _splat(col)])
            i0 = plsc.cumsum(c0)
            i_ref[...] = i0
            e_ref[...] = i0 - c0
            return i0[15]

        pop_pos = prefixes(0, ipos_v, epos_v)
        pop_neg = prefixes(1, ineg_v, eneg_v)

        n_init = (jnp.maximum(pop_pos, 32) + 15) // 16

        def init_body(t, c):
            b = t * 16
            plsc.store_scatter(perm_v, [b + iota], b + iota)
            return c
        lax.fori_loop(0, n_init, init_body, 0)

        def smear(x):
            x = x | (x >> 1)
            x = x | (x >> 2)
            x = x | (x >> 4)
            x = x | (x >> 8)
            x = x | (x >> 16)
            return x

        def ensure(ptr, base):
            def do(_):
                nb = jnp.minimum((ptr // 8) * 8, _SLEN - _SBUF)
                nb = pl.multiple_of(nb, 8)
                pltpu.sync_copy(stream_hbm.at[pl.ds(nb, _SBUF)], sbuf_v)
                return nb
            need = jnp.logical_or(ptr < base, ptr + _MARGIN > base + _SBUF)
            return lax.cond(need, do, lambda _: base, 0)

        def draw(ptr, base, bound_u):
            mask = smear(bound_u)

            def cond(c):
                return c[0] > bound_u

            def body(c):
                _, p = c
                off = jnp.minimum(p - base, _SBUF - 1)
                w = _sread1(sbuf_v, off).astype(jnp.uint32)
                return (w & mask, p + 1)

            v, ptr = lax.while_loop(cond, body, (bound_u + jnp.uint32(1), ptr))
            return v, ptr

        lane01 = iota < 2

        def fy_body(d, carry):
            ptr, base = carry
            i = pop_pos - 1 - d
            base = ensure(ptr, base)
            j_u, ptr = draw(ptr, base, i.astype(jnp.uint32))
            j = j_u.astype(jnp.int32)
            ij = jnp.where(iota == 0, i, j)
            pij = plsc.load_gather(perm_v, [ij])
            pi = pij[0]
            pj = pij[1]
            plsc.store_scatter(perm_v, [ij],
                               jnp.where(iota == 0, pj, pi), mask=lane01)
            return (ptr, base)

        _, base_end = lax.fori_loop(
            0, jnp.maximum(pop_pos - 1, 0), fy_body,
            (jnp.int32(0), jnp.int32(0)))

        def vlookup(list_ref, i_ref, e_ref, r):
            w = jnp.zeros((16,), jnp.int32)
            for step in (8, 4, 2, 1):
                probe = jnp.minimum(w + (step - 1), _NSUB - 1)
                ipv = plsc.load_gather(i_ref, [probe])
                w = jnp.where(ipv <= r, w + step, w)
            w = jnp.minimum(w, _NSUB - 1)
            off = jnp.clip(r - plsc.load_gather(e_ref, [w]), 0, _CHUNK2 - 1)
            return plsc.load_gather(list_ref, [w, off])

        for c in range(2):
            r = perm_v[pl.ds(c * 16, 16)]
            pidx = vlookup(posl_v, ipos_v, epos_v, r)
            nidx = vlookup(negl_v, ineg_v, eneg_v, r - pop_pos)
            keep_v[pl.ds(c * 16, 16)] = jnp.where(r < pop_pos, pidx, nidx)

        bound_n = (pop_neg - 1).astype(jnp.uint32)
        maskb = smear(bound_n)

        def ncond(c):
            return c[0] < 96

        def nbody(c):
            cnt, ptr, base = c
            base = ensure(ptr, base)
            off = jnp.minimum(ptr - base, _SBUF - 16)
            w = plsc.bitcast(plsc.load_gather(sbuf_v, [off + iota]),
                             jnp.uint32)
            wm = w & maskb
            ok = wm <= bound_n
            oki = ok.astype(jnp.int32)
            pos = cnt + plsc.cumsum(oki) - 1
            plsc.store_scatter(
                keep_v, [jnp.clip(32 + pos, 32, 127)],
                plsc.bitcast(wm, jnp.int32),
                mask=jnp.logical_and(ok, pos < 96))
            return (cnt + jnp.sum(oki), ptr + 16, base)

        lax.while_loop(ncond, nbody, (jnp.int32(0), jnp.int32(0), base_end))

        for c in range(2, 8):
            d = keep_v[pl.ds(c * 16, 16)]
            idx = vlookup(negl_v, ineg_v, eneg_v, d)
            keep_v[pl.ds(c * 16, 16)] = jnp.where(
                _splat(pop_neg) == 0, 0, idx)

        pltpu.async_copy(rois16_hbm.at[keep_v], roisg_v, sem).wait()
        for c in range(8):
            rows = c * 16 + iota
            kvec = keep_v[pl.ds(c * 16, 16)]
            px1 = plsc.load_gather(roisg_v, [rows, _splat(0)])
            py1 = plsc.load_gather(roisg_v, [rows, _splat(1)])
            px2 = plsc.load_gather(roisg_v, [rows, _splat(2)])
            py2 = plsc.load_gather(roisg_v, [rows, _splat(3)])
            avec = plsc.load_gather(amax_v, [kvec])
            lvec = plsc.load_gather(label_v, [avec])
            if c < 2:
                cls_v[pl.ds(c * 16, 16)] = lvec + 1
            else:
                cls_v[pl.ds(c * 16, 16)] = jnp.zeros((16,), jnp.int32)
            gx1 = plsc.load_gather(bbox_v, [avec, _splat(0)])
            gy1 = plsc.load_gather(bbox_v, [avec, _splat(1)])
            gx2 = plsc.load_gather(bbox_v, [avec, _splat(2)])
            gy2 = plsc.load_gather(bbox_v, [avec, _splat(3)])
            pw = px2 - px1
            ph = py2 - py1
            tx = ((gx2 + gx1) / 2.0 - (px2 + px1) / 2.0) / pw
            ty = ((gy2 + gy1) / 2.0 - (py2 + py1) / 2.0) / ph
            tw = _ln((gx2 - gx1) / pw)
            th = _ln((gy2 - gy1) / ph)
            for col, vec in ((0, px1), (1, py1), (2, px2), (3, py2)):
                plsc.store_scatter(srois_v, [rows, _splat(col)], vec)
            for col, vec in ((0, tx), (1, ty), (2, tw), (3, th)):
                plsc.store_scatter(reg_v, [rows, _splat(col)], vec)

        pltpu.sync_copy(cls_v, cls_hbm)
        pltpu.sync_copy(reg_v, reg_hbm)
        pltpu.sync_copy(srois_v, srois_hbm)


def kernel(bbox, label, rois):
    bbox0 = bbox[0]
    label0 = label[0]
    rois_cat = jnp.concatenate([rois, bbox0], axis=0)
    rois_t = jnp.pad(rois_cat, ((0, _NPAD - _N), (0, 0))).T
    iou_max2, amax2 = _iou_call(rois_t, bbox0)
    iou_max = iou_max2.reshape(_NPAD)
    amax = amax2.reshape(_NPAD)
    rois16 = jnp.pad(rois_cat, ((0, 0), (0, 12)))
    stream = jnp.asarray(_STREAM_NP)
    cls, reg, sample_rois = _fused_call(
        iou_max, amax, stream, rois16, bbox0, label0)
    return (cls, reg, sample_rois)
